# baseline (speedup 1.0000x reference)
import functools

import jax
import jax.numpy as jnp
import numpy as np
from jax import lax
from jax.experimental import pallas as pl
from jax.experimental.pallas import tpu as pltpu
from jax.experimental.pallas import tpu_sc as plsc

_H = 256
_W = 256
_A = 180
_D = 256
_B = 4
_P = _H * _W

_NC = 2
_NS = 16
_NWORK = _NC * _NS

_KMAX = 10112
_KSTEPS = _KMAX // 128
_SPAN = 16
_GROWS = 200
_IMG_PAD = 66048
_ZCHUNK = _IMG_PAD // _NS
_OCHUNK = _P // _NS


def _filter_matrix() -> np.ndarray:
    n = np.arange(_D)
    f = np.zeros(_D, dtype=np.float64)
    f[0] = 0.25
    f[1::2] = -1.0 / (np.pi ** 2 * n[1::2].astype(np.float64) ** 2)
    eye = np.eye(_D, dtype=np.float64)
    C = np.fft.ifft(np.fft.fft(eye, axis=1) * f[None, :], axis=1).real
    C *= np.pi / _A
    return C.astype(np.float32)


def _index_tables():
    angles = np.linspace(0.0, np.pi, _A).astype(np.float32)
    cos = np.cos(angles).astype(np.float32)
    sin = np.sin(angles).astype(np.float32)
    y, x = np.meshgrid(np.arange(_H), np.arange(_W), indexing='ij')
    xc = (x - _W / 2).astype(np.float32)
    yc = (y - _H / 2).astype(np.float32)
    rot = xc[None] * cos[:, None, None] + yc[None] * sin[:, None, None]
    idx = np.clip((rot / (2 * np.pi) * _D).astype(np.int32), 0, _D - 1)
    idx = idx.reshape(_A, _P)

    L = idx == 0
    band = (idx >= 1) & (idx <= 254)

    padl = np.zeros((_A + 2, _P), np.int8)
    padl[1:_A + 1] = L
    dl = np.diff(padl, axis=0)
    q = np.zeros((_NWORK, 4, _P // _NWORK), np.int32)
    lo = np.zeros((_P, 2), np.int32)
    hi = np.zeros((_P, 2), np.int32)
    kk = np.zeros(_P, np.int32)
    pr, ar = np.nonzero(dl.T == 1)
    for p, a in zip(pr, ar):
        lo[p, kk[p]] = a
        kk[p] += 1
    kk[:] = 0
    pr, ar = np.nonzero(dl.T == -1)
    for p, a in zip(pr, ar):
        hi[p, kk[p]] = a
        kk[p] += 1
    ppw = _P // _NWORK
    for w in range(_NWORK):
        sl = slice(w * ppw, (w + 1) * ppw)
        q[w, 0] = hi[sl, 0]
        q[w, 1] = lo[sl, 0]
        q[w, 2] = hi[sl, 1]
        q[w, 3] = lo[sl, 1]

    aa, pp = np.nonzero(band)
    tot = len(aa)
    starts = [round(tot * w / _NWORK) for w in range(_NWORK + 1)]

    lf = np.zeros((_NWORK, _KSTEPS, 128), np.int32)
    pix = np.zeros((_NWORK, _KSTEPS, 128), np.int32)
    a0s = np.zeros(_NWORK, np.int32)
    for w in range(_NWORK):
        s, e = starts[w], starts[w + 1]
        c = e - s
        a0 = int(aa[s] // 8) * 8
        assert c <= _KMAX and int(aa[e - 1]) - a0 < _SPAN
        a0s[w] = a0
        lf[w].reshape(_KMAX)[:c] = (aa[s:e] - a0) * _D + idx[aa[s:e], pp[s:e]]
        pw = pix[w].reshape(_KMAX)
        pw[:c] = pp[s:e]
        pw[c:] = _P + (np.arange(_KMAX - c) % 512)
    return q, a0s, lf, pix


_C_MAT = _filter_matrix()
_QIV, _A0S, _LF, _PIX = _index_tables()
_PPW = _P // _NWORK
_CT_ROWS = 184


def _filter_body(x_ref, c_ref, g_ref, ct_ref):
    fm = jnp.dot(x_ref[...].reshape(_B * _A, _D), c_ref[...],
                 preferred_element_type=jnp.float32)
    ii = jax.lax.broadcasted_iota(jnp.int32, (_CT_ROWS, _A), 0)
    aa = jax.lax.broadcasted_iota(jnp.int32, (_CT_ROWS, _A), 1)
    tri = ((aa < ii) & (ii <= _A)).astype(jnp.float32)
    sel = (ii == _A + 1).astype(jnp.float32)
    for b in range(_B):
        blk = fm[b * _A:(b + 1) * _A, :]
        g_ref[b * _GROWS: b * _GROWS + _A, :] = blk - blk[:, 255:256]
        g_ref[b * _GROWS + _A: (b + 1) * _GROWS, :] = jnp.zeros(
            (_GROWS - _A, _D), jnp.float32)
        fd0col = blk[:, 0:1] - blk[:, 255:256]
        f255col = blk[:, 255:256]
        ct_ref[b * _CT_ROWS:(b + 1) * _CT_ROWS, :] = (
            jnp.dot(tri, fd0col, preferred_element_type=jnp.float32)
            + jnp.dot(sel, f255col, preferred_element_type=jnp.float32))


_FILT_CALL = pl.pallas_call(
    _filter_body,
    out_shape=(
        jax.ShapeDtypeStruct((_B * _GROWS, _D), jnp.float32),
        jax.ShapeDtypeStruct((_B * _CT_ROWS, 1), jnp.float32),
    ),
)


def _band_body(g_hbm, ct_hbm, lf_hbm, pix_hbm, q_hbm, zero_hbm, dummy_hbm,
               out_hbm, out2_hbm,
               tbl0, tbl1, tbl2, tbl3, ct0, ct1, ct2, ct3,
               lf_v, pix_v, q_v,
               vals0, vals1, vals2, vals3,
               dbuf0, dbuf1, dbuf2, dbuf3,
               sem, semz, semd, semt0, semt1, semt2, semt3,
               img0, img1, img2, img3):
    cid = lax.axis_index("c")
    sid = lax.axis_index("s")
    wid = cid * _NS + sid
    imgs = [img0, img1, img2, img3]
    vals = [vals0, vals1, vals2, vals3]
    tbls = [tbl0, tbl1, tbl2, tbl3]
    cts = [ct0, ct1, ct2, ct3]
    dbufs = [dbuf0, dbuf1, dbuf2, dbuf3]
    semts = [semt0, semt1, semt2, semt3]

    a0 = jnp.int32(0)
    for w in range(_NWORK):
        a0 = a0 + jnp.where(wid == w, jnp.int32(int(_A0S[w])), jnp.int32(0))

    with jax.named_scope("bp_fire"):
        for b in range(_B):
            pltpu.async_copy(zero_hbm,
                             imgs[b].at[pl.ds(sid * _ZCHUNK, _ZCHUNK)], semz)
            row0 = pl.multiple_of(b * _GROWS + a0, 8)
            pltpu.async_copy(g_hbm.at[pl.ds(row0, _SPAN)], tbls[b], semts[b])

    with jax.named_scope("bp_stage"):
        pltpu.sync_copy(lf_hbm.at[wid], lf_v)
        pltpu.sync_copy(pix_hbm.at[wid], pix_v)
        pltpu.sync_copy(q_hbm.at[wid], q_v)
        for b in range(_B):
            pltpu.sync_copy(ct_hbm.at[pl.ds(b * _CT_ROWS, _CT_ROWS)], cts[b])

    with jax.named_scope("bp_zwait"):
        for b in range(_B):
            pltpu.make_async_copy(
                zero_hbm, imgs[b].at[pl.ds(sid * _ZCHUNK, _ZCHUNK)],
                semz).wait()
    plsc.subcore_barrier()

    for b in range(_B):
        vals_v = vals[b]
        tbl_v = tbls[b]
        with jax.named_scope("bp_twait"):
            row0 = pl.multiple_of(b * _GROWS + a0, 8)
            pltpu.make_async_copy(g_hbm.at[pl.ds(row0, _SPAN)], tbl_v,
                                  semts[b]).wait()

        with jax.named_scope("bp_gather"):
            @plsc.parallel_loop(0, _KSTEPS, unroll=2)
            def gstep(j):
                for l in range(8):
                    iv = lf_v[j, pl.ds(l * 16, 16)]
                    rv = lax.shift_right_logical(iv, 8)
                    cv = lax.bitwise_and(iv, 255)
                    vals_v[j, pl.ds(l * 16, 16)] = plsc.load_gather(
                        tbl_v, [rv, cv])

        with jax.named_scope("bp_scatter"):
            def sstep(j, carry):
                pltpu.async_copy(vals_v.at[j], imgs[b].at[pix_v.at[j]], sem,
                                 add=True)
                return carry

            lax.fori_loop(0, _KSTEPS, sstep, jnp.int32(0))

        with jax.named_scope("bp_dense"):
            ct_b = cts[b]
            dbuf_b = dbufs[b]
            zv = jnp.zeros((16,), jnp.int32)
            c181 = jnp.full((16,), _A + 1, jnp.int32)
            s255v = plsc.load_gather(ct_b, [c181, zv])

            @plsc.parallel_loop(0, _PPW // 16, unroll=2)
            def dstep(i):
                s = pl.ds(i * 16, 16)
                h1 = q_v[0, s]
                l1 = q_v[1, s]
                h2 = q_v[2, s]
                l2 = q_v[3, s]
                dbuf_b[s] = (plsc.load_gather(ct_b, [h1, zv])
                             - plsc.load_gather(ct_b, [l1, zv])
                             + plsc.load_gather(ct_b, [h2, zv])
                             - plsc.load_gather(ct_b, [l2, zv])
                             + s255v)

            pltpu.async_copy(
                dbuf_b, out2_hbm.at[pl.ds(b * _P + wid * _PPW, _PPW)], semd)

    with jax.named_scope("bp_drain"):
        for b in range(_B):
            pltpu.make_async_copy(dummy_hbm, vals[b], sem).wait()
            pltpu.make_async_copy(zero_hbm.at[pl.ds(0, _PPW)], dbufs[b],
                                  semd).wait()

    plsc.subcore_barrier()

    with jax.named_scope("bp_out"):
        for b in range(_B):
            pltpu.sync_copy(
                imgs[b].at[pl.ds(sid * _OCHUNK, _OCHUNK)],
                out_hbm.at[pl.ds(cid * (_B * _P) + b * _P + sid * _OCHUNK,
                                 _OCHUNK)])


@functools.cache
def _band_call():
  return pl.kernel(
    _band_body,
    out_type=(jax.ShapeDtypeStruct((_NC * _B * _P,), jnp.float32),
              jax.ShapeDtypeStruct((_B * _P,), jnp.float32)),
    mesh=plsc.VectorSubcoreMesh(core_axis_name="c", subcore_axis_name="s",
                                num_cores=_NC, num_subcores=_NS),
    scratch_types=[
        pltpu.VMEM((_SPAN, _D), jnp.float32),
        pltpu.VMEM((_SPAN, _D), jnp.float32),
        pltpu.VMEM((_SPAN, _D), jnp.float32),
        pltpu.VMEM((_SPAN, _D), jnp.float32),
        pltpu.VMEM((_CT_ROWS, 1), jnp.float32),
        pltpu.VMEM((_CT_ROWS, 1), jnp.float32),
        pltpu.VMEM((_CT_ROWS, 1), jnp.float32),
        pltpu.VMEM((_CT_ROWS, 1), jnp.float32),
        pltpu.VMEM((_KSTEPS, 128), jnp.int32),
        pltpu.VMEM((_KSTEPS, 128), jnp.int32),
        pltpu.VMEM((4, _PPW), jnp.int32),
        pltpu.VMEM((_KSTEPS, 128), jnp.float32),
        pltpu.VMEM((_KSTEPS, 128), jnp.float32),
        pltpu.VMEM((_KSTEPS, 128), jnp.float32),
        pltpu.VMEM((_KSTEPS, 128), jnp.float32),
        pltpu.VMEM((_PPW,), jnp.float32),
        pltpu.VMEM((_PPW,), jnp.float32),
        pltpu.VMEM((_PPW,), jnp.float32),
        pltpu.VMEM((_PPW,), jnp.float32),
        pltpu.SemaphoreType.DMA,
        pltpu.SemaphoreType.DMA,
        pltpu.SemaphoreType.DMA,
        pltpu.SemaphoreType.DMA,
        pltpu.SemaphoreType.DMA,
        pltpu.SemaphoreType.DMA,
        pltpu.SemaphoreType.DMA,
        pltpu.VMEM_SHARED((_IMG_PAD,), jnp.float32),
        pltpu.VMEM_SHARED((_IMG_PAD,), jnp.float32),
        pltpu.VMEM_SHARED((_IMG_PAD,), jnp.float32),
        pltpu.VMEM_SHARED((_IMG_PAD,), jnp.float32),
    ],
    compiler_params=pltpu.CompilerParams(use_tc_tiling_on_sc=False,
                                         needs_layout_passes=False),
  )


@jax.jit
def kernel(sinograms):
    g, ct = _FILT_CALL(sinograms, jnp.asarray(_C_MAT))

    band, dense = _band_call()(
        g, ct,
        jnp.asarray(_LF), jnp.asarray(_PIX), jnp.asarray(_QIV),
        jnp.zeros((_ZCHUNK,), jnp.float32),
        jnp.zeros((_KSTEPS, 128), jnp.float32))

    band = band.reshape(_NC, _B, _P)
    rec = (band[0] + band[1] + dense.reshape(_B, _P)).reshape(_B, _H, _W)
    return jnp.clip(rec, 0.0, rec.max())


# speedup vs baseline: 1.0577x; 1.0577x over previous
import functools

import jax
import jax.numpy as jnp
import numpy as np
from jax import lax
from jax.experimental import pallas as pl
from jax.experimental.pallas import tpu as pltpu
from jax.experimental.pallas import tpu_sc as plsc

_H = 256
_W = 256
_A = 180
_D = 256
_B = 4
_P = _H * _W

_NC = 2
_NS = 16
_NWORK = _NC * _NS

_KMAX = 10112
_KSTEPS = _KMAX // 128
_SPAN = 16
_GROWS = 200
_IMG_PAD = 66048
_ZCHUNK = _IMG_PAD // _NS
_OCHUNK = _P // _NS

_A_PAD = 192
_PIX_BLK = 2048
_N_BLOCKS = _P // _PIX_BLK


def _filter_matrix() -> np.ndarray:
    n = np.arange(_D)
    f = np.zeros(_D, dtype=np.float64)
    f[0] = 0.25
    f[1::2] = -1.0 / (np.pi ** 2 * n[1::2].astype(np.float64) ** 2)
    eye = np.eye(_D, dtype=np.float64)
    C = np.fft.ifft(np.fft.fft(eye, axis=1) * f[None, :], axis=1).real
    C *= np.pi / _A
    return C.astype(np.float32)


def _index_tables():
    angles = np.linspace(0.0, np.pi, _A).astype(np.float32)
    cos = np.cos(angles).astype(np.float32)
    sin = np.sin(angles).astype(np.float32)
    y, x = np.meshgrid(np.arange(_H), np.arange(_W), indexing='ij')
    xc = (x - _W / 2).astype(np.float32)
    yc = (y - _H / 2).astype(np.float32)
    rot = xc[None] * cos[:, None, None] + yc[None] * sin[:, None, None]
    idx = np.clip((rot / (2 * np.pi) * _D).astype(np.int32), 0, _D - 1)
    idx = idx.reshape(_A, _P)

    lmat = np.zeros((_A_PAD, _P), np.float16)
    lmat[:_A] = (idx == 0)
    lmat = lmat.astype(jnp.bfloat16)
    band = (idx >= 1) & (idx <= 254)

    aa, pp = np.nonzero(band)
    tot = len(aa)
    starts = [round(tot * w / _NWORK) for w in range(_NWORK + 1)]

    lf = np.zeros((_NWORK, _KSTEPS, 128), np.int32)
    pix = np.zeros((_NWORK, _KSTEPS, 128), np.int32)
    a0s = np.zeros(_NWORK, np.int32)
    for w in range(_NWORK):
        s, e = starts[w], starts[w + 1]
        c = e - s
        a0 = int(aa[s] // 8) * 8
        assert c <= _KMAX and int(aa[e - 1]) - a0 < _SPAN
        a0s[w] = a0
        lf[w].reshape(_KMAX)[:c] = (aa[s:e] - a0) * _D + idx[aa[s:e], pp[s:e]]
        pw = pix[w].reshape(_KMAX)
        pw[:c] = pp[s:e]
        pw[c:] = _P + (np.arange(_KMAX - c) % 512)
    return lmat, a0s, lf, pix


_C_MAT = _filter_matrix()
_LMAT, _A0S, _LF, _PIX = _index_tables()


def _filter_body(x_ref, c_ref, sel_ref, g_ref, fd_ref):
    fm = jnp.dot(x_ref[...].reshape(_B * _A, _D), c_ref[...],
                 preferred_element_type=jnp.float32)
    for b in range(_B):
        blk = fm[b * _A:(b + 1) * _A, :]
        g_ref[b * _GROWS: b * _GROWS + _A, :] = blk - blk[:, 255:256]
        g_ref[b * _GROWS + _A: (b + 1) * _GROWS, :] = jnp.zeros(
            (_GROWS - _A, _D), jnp.float32)
        fd_ref[b * _A_PAD: b * _A_PAD + _A, :] = jnp.dot(
            blk, sel_ref[...], preferred_element_type=jnp.float32)
        fd_ref[b * _A_PAD + _A: (b + 1) * _A_PAD, :] = jnp.zeros(
            (_A_PAD - _A, 2), jnp.float32)


_SEL = np.zeros((_D, 2), np.float32)
_SEL[0, 0] = 1.0
_SEL[255, 0] = -1.0
_SEL[255, 1] = 1.0

_ZER = np.zeros((_ZCHUNK,), np.float32)
_DUM = np.zeros((_KSTEPS, 128), np.float32)

_FILT_CALL = pl.pallas_call(
    _filter_body,
    out_shape=(
        jax.ShapeDtypeStruct((_B * _GROWS, _D), jnp.float32),
        jax.ShapeDtypeStruct((_B * _A_PAD, 2), jnp.float32),
    ),
)


def _dense_body(fd0_ref, f255_ref, l_ref, o_ref):
    lmat = l_ref[...].astype(jnp.float32)

    s255 = jnp.sum(f255_ref[...], axis=1, keepdims=True)
    o_ref[...] = s255 + jnp.dot(fd0_ref[...], lmat,
                                preferred_element_type=jnp.float32)


_DENSE_CALL = pl.pallas_call(
    _dense_body,
    grid=(_N_BLOCKS,),
    in_specs=[
        pl.BlockSpec((_B, _A_PAD), lambda i: (0, 0)),
        pl.BlockSpec((_B, _A_PAD), lambda i: (0, 0)),
        pl.BlockSpec((_A_PAD, _PIX_BLK), lambda i: (0, i)),
    ],
    out_specs=pl.BlockSpec((_B, _PIX_BLK), lambda i: (0, i)),
    out_shape=jax.ShapeDtypeStruct((_B, _P), jnp.float32),
)


def _comb_body(band_ref, dense_ref, o_ref):
    rec = band_ref[0:_B, :] + band_ref[_B:2 * _B, :] + dense_ref[...]
    o_ref[...] = jnp.clip(rec, 0.0, jnp.max(rec))


_COMB_CALL = pl.pallas_call(
    _comb_body,
    out_shape=jax.ShapeDtypeStruct((_B, _P), jnp.float32),
)


def _band_body(g_hbm, lf_hbm, pix_hbm, zero_hbm, dummy_hbm,
               out_hbm,
               tbl0, tbl1, tbl2, tbl3, lf_v, pix_v,
               vals0, vals1, vals2, vals3,
               sem, semz, semt0, semt1, semt2, semt3,
               img0, img1, img2, img3):
    cid = lax.axis_index("c")
    sid = lax.axis_index("s")
    wid = cid * _NS + sid
    imgs = [img0, img1, img2, img3]
    vals = [vals0, vals1, vals2, vals3]
    tbls = [tbl0, tbl1, tbl2, tbl3]
    semts = [semt0, semt1, semt2, semt3]

    a0 = jnp.int32(0)
    for w in range(_NWORK):
        a0 = a0 + jnp.where(wid == w, jnp.int32(int(_A0S[w])), jnp.int32(0))

    with jax.named_scope("bp_fire"):
        for b in range(_B):
            pltpu.async_copy(zero_hbm,
                             imgs[b].at[pl.ds(sid * _ZCHUNK, _ZCHUNK)], semz)
            row0 = pl.multiple_of(b * _GROWS + a0, 8)
            pltpu.async_copy(g_hbm.at[pl.ds(row0, _SPAN)], tbls[b], semts[b])

    with jax.named_scope("bp_stage"):
        pltpu.sync_copy(lf_hbm.at[wid], lf_v)
        pltpu.sync_copy(pix_hbm.at[wid], pix_v)

    with jax.named_scope("bp_zwait"):
        for b in range(_B):
            pltpu.make_async_copy(
                zero_hbm, imgs[b].at[pl.ds(sid * _ZCHUNK, _ZCHUNK)],
                semz).wait()
    plsc.subcore_barrier()

    for b in range(_B):
        vals_v = vals[b]
        tbl_v = tbls[b]
        with jax.named_scope("bp_twait"):
            row0 = pl.multiple_of(b * _GROWS + a0, 8)
            pltpu.make_async_copy(g_hbm.at[pl.ds(row0, _SPAN)], tbl_v,
                                  semts[b]).wait()

        with jax.named_scope("bp_gather"):
            @plsc.parallel_loop(0, _KSTEPS, unroll=2)
            def gstep(j):
                for l in range(8):
                    iv = lf_v[j, pl.ds(l * 16, 16)]
                    rv = lax.shift_right_logical(iv, 8)
                    cv = lax.bitwise_and(iv, 255)
                    vals_v[j, pl.ds(l * 16, 16)] = plsc.load_gather(
                        tbl_v, [rv, cv])

        with jax.named_scope("bp_scatter"):
            def sstep(j, carry):
                pltpu.async_copy(vals_v.at[j], imgs[b].at[pix_v.at[j]], sem,
                                 add=True)
                return carry

            lax.fori_loop(0, _KSTEPS, sstep, jnp.int32(0))

    with jax.named_scope("bp_drain"):
        for b in range(_B):
            pltpu.make_async_copy(dummy_hbm, vals[b], sem).wait()

    plsc.subcore_barrier()

    with jax.named_scope("bp_out"):
        for b in range(_B):
            pltpu.sync_copy(
                imgs[b].at[pl.ds(sid * _OCHUNK, _OCHUNK)],
                out_hbm.at[cid * _B + b, pl.ds(sid * _OCHUNK, _OCHUNK)])


@functools.cache
def _band_call():
  return pl.kernel(
    _band_body,
    out_type=jax.ShapeDtypeStruct((_NC * _B, _P), jnp.float32),
    mesh=plsc.VectorSubcoreMesh(core_axis_name="c", subcore_axis_name="s",
                                num_cores=_NC, num_subcores=_NS),
    scratch_types=[
        pltpu.VMEM((_SPAN, _D), jnp.float32),
        pltpu.VMEM((_SPAN, _D), jnp.float32),
        pltpu.VMEM((_SPAN, _D), jnp.float32),
        pltpu.VMEM((_SPAN, _D), jnp.float32),
        pltpu.VMEM((_KSTEPS, 128), jnp.int32),
        pltpu.VMEM((_KSTEPS, 128), jnp.int32),
        pltpu.VMEM((_KSTEPS, 128), jnp.float32),
        pltpu.VMEM((_KSTEPS, 128), jnp.float32),
        pltpu.VMEM((_KSTEPS, 128), jnp.float32),
        pltpu.VMEM((_KSTEPS, 128), jnp.float32),
        pltpu.SemaphoreType.DMA,
        pltpu.SemaphoreType.DMA,
        pltpu.SemaphoreType.DMA,
        pltpu.SemaphoreType.DMA,
        pltpu.SemaphoreType.DMA,
        pltpu.SemaphoreType.DMA,
        pltpu.VMEM_SHARED((_IMG_PAD,), jnp.float32),
        pltpu.VMEM_SHARED((_IMG_PAD,), jnp.float32),
        pltpu.VMEM_SHARED((_IMG_PAD,), jnp.float32),
        pltpu.VMEM_SHARED((_IMG_PAD,), jnp.float32),
    ],
    compiler_params=pltpu.CompilerParams(use_tc_tiling_on_sc=False,
                                         needs_layout_passes=False),
  )


@jax.jit
def kernel(sinograms):
    g, fd = _FILT_CALL(sinograms, jnp.asarray(_C_MAT), jnp.asarray(_SEL))

    fd0 = fd[:, 0].reshape(_B, _A_PAD)
    f255 = fd[:, 1].reshape(_B, _A_PAD)
    dense = _DENSE_CALL(fd0, f255, jnp.asarray(_LMAT))

    band = _band_call()(
        g,
        jnp.asarray(_LF), jnp.asarray(_PIX),
        jnp.asarray(_ZER), jnp.asarray(_DUM))

    return _COMB_CALL(band, dense).reshape(_B, _H, _W)


# speedup vs baseline: 1.0881x; 1.0288x over previous
import functools

import jax
import jax.numpy as jnp
import numpy as np
from jax import lax
from jax.experimental import pallas as pl
from jax.experimental.pallas import tpu as pltpu
from jax.experimental.pallas import tpu_sc as plsc

_H = 256
_W = 256
_A = 180
_D = 256
_B = 4
_P = _H * _W

_NC = 2
_NS = 16
_NWORK = _NC * _NS

_KMAX = 10112
_KSTEPS = _KMAX // 128
_SPAN = 16
_GROWS = 200
_IMG_PAD = 66048
_ZCHUNK = _IMG_PAD // _NS
_OCHUNK = _P // _NS

_A_PAD = 192
_PIX_BLK = 2048
_N_BLOCKS = _P // _PIX_BLK


def _filter_matrix() -> np.ndarray:
    n = np.arange(_D)
    f = np.zeros(_D, dtype=np.float64)
    f[0] = 0.25
    f[1::2] = -1.0 / (np.pi ** 2 * n[1::2].astype(np.float64) ** 2)
    eye = np.eye(_D, dtype=np.float64)
    C = np.fft.ifft(np.fft.fft(eye, axis=1) * f[None, :], axis=1).real
    C *= np.pi / _A
    return C.astype(np.float32)


def _index_tables():
    angles = np.linspace(0.0, np.pi, _A).astype(np.float32)
    cos = np.cos(angles).astype(np.float32)
    sin = np.sin(angles).astype(np.float32)
    y, x = np.meshgrid(np.arange(_H), np.arange(_W), indexing='ij')
    xc = (x - _W / 2).astype(np.float32)
    yc = (y - _H / 2).astype(np.float32)
    rot = xc[None] * cos[:, None, None] + yc[None] * sin[:, None, None]
    idx = np.clip((rot / (2 * np.pi) * _D).astype(np.int32), 0, _D - 1)
    idx = idx.reshape(_A, _P)

    lmat = np.zeros((_A_PAD, _P), np.float16)
    lmat[:_A] = (idx == 0)
    lmat = lmat.astype(jnp.bfloat16)
    band = (idx >= 1) & (idx <= 254)

    aa, pp = np.nonzero(band)
    tot = len(aa)
    starts = [round(tot * w / _NWORK) for w in range(_NWORK + 1)]

    lf = np.zeros((_NWORK, _KSTEPS, 128), np.int32)
    pix = np.zeros((_NWORK, _KSTEPS, 128), np.int32)
    a0s = np.zeros(_NWORK, np.int32)
    for w in range(_NWORK):
        s, e = starts[w], starts[w + 1]
        c = e - s
        a0 = int(aa[s] // 8) * 8
        assert c <= _KMAX and int(aa[e - 1]) - a0 < _SPAN
        a0s[w] = a0
        lf[w].reshape(_KMAX)[:c] = (aa[s:e] - a0) * _D + idx[aa[s:e], pp[s:e]]
        pw = pix[w].reshape(_KMAX)
        pw[:c] = pp[s:e]
        pw[c:] = _P + (np.arange(_KMAX - c) % 512)
    return lmat, a0s, lf, pix


_C_MAT = _filter_matrix()
_LMAT, _A0S, _LF, _PIX = _index_tables()


def _filter_body(x_ref, c_ref, sel_ref, g_ref, fd_ref):
    fm = jnp.dot(x_ref[...].reshape(_B * _A, _D), c_ref[...],
                 preferred_element_type=jnp.float32)
    for b in range(_B):
        blk = fm[b * _A:(b + 1) * _A, :]
        g_ref[b * _GROWS: b * _GROWS + _A, :] = blk - blk[:, 255:256]
        g_ref[b * _GROWS + _A: (b + 1) * _GROWS, :] = jnp.zeros(
            (_GROWS - _A, _D), jnp.float32)
        fd_ref[b * _A_PAD: b * _A_PAD + _A, :] = jnp.dot(
            blk, sel_ref[...], preferred_element_type=jnp.float32)
        fd_ref[b * _A_PAD + _A: (b + 1) * _A_PAD, :] = jnp.zeros(
            (_A_PAD - _A, 2), jnp.float32)


_SEL = np.zeros((_D, 2), np.float32)
_SEL[0, 0] = 1.0
_SEL[255, 0] = -1.0
_SEL[255, 1] = 1.0

_ZER = np.zeros((_ZCHUNK,), np.float32)
_DUM = np.zeros((_KSTEPS, 128), np.float32)

_FILT_CALL = pl.pallas_call(
    _filter_body,
    out_shape=(
        jax.ShapeDtypeStruct((_B * _GROWS, _D), jnp.float32),
        jax.ShapeDtypeStruct((_B * _A_PAD, 2), jnp.float32),
    ),
)


def _dense_body(fd0_ref, f255_ref, l_ref, o_ref):
    lmat = l_ref[...].astype(jnp.float32)

    s255 = jnp.sum(f255_ref[...], axis=1, keepdims=True)
    o_ref[...] = s255 + jnp.dot(fd0_ref[...], lmat,
                                preferred_element_type=jnp.float32)


_DENSE_CALL = pl.pallas_call(
    _dense_body,
    grid=(_N_BLOCKS,),
    in_specs=[
        pl.BlockSpec((_B, _A_PAD), lambda i: (0, 0)),
        pl.BlockSpec((_B, _A_PAD), lambda i: (0, 0)),
        pl.BlockSpec((_A_PAD, _PIX_BLK), lambda i: (0, i)),
    ],
    out_specs=pl.BlockSpec((_B, _PIX_BLK), lambda i: (0, i)),
    out_shape=jax.ShapeDtypeStruct((_B, _P), jnp.float32),
)


def _band_body(g_hbm, lf_hbm, pix_hbm, zero_hbm, dummy_hbm,
               out_hbm,
               tbl0, tbl1, tbl2, tbl3, lf_v, pix_v,
               vals0, vals1, vals2, vals3,
               sem, semz, semt0, semt1, semt2, semt3,
               img0, img1, img2, img3):
    cid = lax.axis_index("c")
    sid = lax.axis_index("s")
    wid = cid * _NS + sid
    imgs = [img0, img1, img2, img3]
    vals = [vals0, vals1, vals2, vals3]
    tbls = [tbl0, tbl1, tbl2, tbl3]
    semts = [semt0, semt1, semt2, semt3]

    a0 = jnp.int32(0)
    for w in range(_NWORK):
        a0 = a0 + jnp.where(wid == w, jnp.int32(int(_A0S[w])), jnp.int32(0))

    with jax.named_scope("bp_fire"):
        for b in range(_B):
            pltpu.async_copy(zero_hbm,
                             imgs[b].at[pl.ds(sid * _ZCHUNK, _ZCHUNK)], semz)

    with jax.named_scope("bp_stage"):
        pltpu.sync_copy(lf_hbm.at[wid], lf_v)
        pltpu.sync_copy(pix_hbm.at[wid], pix_v)
        for b in range(_B):
            row0 = pl.multiple_of(b * _GROWS + a0, 8)
            pltpu.async_copy(g_hbm.at[pl.ds(row0, _SPAN)], tbls[b], semts[b])

    with jax.named_scope("bp_zwait"):
        for b in range(_B):
            pltpu.make_async_copy(
                zero_hbm, imgs[b].at[pl.ds(sid * _ZCHUNK, _ZCHUNK)],
                semz).wait()
    plsc.subcore_barrier()

    for b in range(_B):
        vals_v = vals[b]
        tbl_v = tbls[b]
        with jax.named_scope("bp_twait"):
            row0 = pl.multiple_of(b * _GROWS + a0, 8)
            pltpu.make_async_copy(g_hbm.at[pl.ds(row0, _SPAN)], tbl_v,
                                  semts[b]).wait()

        with jax.named_scope("bp_gather"):
            @plsc.parallel_loop(0, _KSTEPS, unroll=2)
            def gstep(j):
                for l in range(8):
                    iv = lf_v[j, pl.ds(l * 16, 16)]
                    rv = lax.shift_right_logical(iv, 8)
                    cv = lax.bitwise_and(iv, 255)
                    vals_v[j, pl.ds(l * 16, 16)] = plsc.load_gather(
                        tbl_v, [rv, cv])

        with jax.named_scope("bp_scatter"):
            def sstep(j, carry):
                pltpu.async_copy(vals_v.at[j], imgs[b].at[pix_v.at[j]], sem,
                                 add=True)
                return carry

            lax.fori_loop(0, _KSTEPS, sstep, jnp.int32(0))

    with jax.named_scope("bp_drain"):
        for b in range(_B):
            pltpu.make_async_copy(dummy_hbm, vals[b], sem).wait()

    plsc.subcore_barrier()

    with jax.named_scope("bp_out"):
        for b in range(_B):
            pltpu.sync_copy(
                imgs[b].at[pl.ds(sid * _OCHUNK, _OCHUNK)],
                out_hbm.at[pl.ds(cid * (_B * _P) + b * _P + sid * _OCHUNK,
                                 _OCHUNK)])


@functools.cache
def _band_call():
  return pl.kernel(
    _band_body,
    out_type=jax.ShapeDtypeStruct((_NC * _B * _P,), jnp.float32),
    mesh=plsc.VectorSubcoreMesh(core_axis_name="c", subcore_axis_name="s",
                                num_cores=_NC, num_subcores=_NS),
    scratch_types=[
        pltpu.VMEM((_SPAN, _D), jnp.float32),
        pltpu.VMEM((_SPAN, _D), jnp.float32),
        pltpu.VMEM((_SPAN, _D), jnp.float32),
        pltpu.VMEM((_SPAN, _D), jnp.float32),
        pltpu.VMEM((_KSTEPS, 128), jnp.int32),
        pltpu.VMEM((_KSTEPS, 128), jnp.int32),
        pltpu.VMEM((_KSTEPS, 128), jnp.float32),
        pltpu.VMEM((_KSTEPS, 128), jnp.float32),
        pltpu.VMEM((_KSTEPS, 128), jnp.float32),
        pltpu.VMEM((_KSTEPS, 128), jnp.float32),
        pltpu.SemaphoreType.DMA,
        pltpu.SemaphoreType.DMA,
        pltpu.SemaphoreType.DMA,
        pltpu.SemaphoreType.DMA,
        pltpu.SemaphoreType.DMA,
        pltpu.SemaphoreType.DMA,
        pltpu.VMEM_SHARED((_IMG_PAD,), jnp.float32),
        pltpu.VMEM_SHARED((_IMG_PAD,), jnp.float32),
        pltpu.VMEM_SHARED((_IMG_PAD,), jnp.float32),
        pltpu.VMEM_SHARED((_IMG_PAD,), jnp.float32),
    ],
    compiler_params=pltpu.CompilerParams(use_tc_tiling_on_sc=False,
                                         needs_layout_passes=False),
  )


@jax.jit
def kernel(sinograms):
    g, fd = _FILT_CALL(sinograms, jnp.asarray(_C_MAT), jnp.asarray(_SEL))

    fd0 = fd[:, 0].reshape(_B, _A_PAD)
    f255 = fd[:, 1].reshape(_B, _A_PAD)
    dense = _DENSE_CALL(fd0, f255, jnp.asarray(_LMAT))

    band = _band_call()(
        g,
        jnp.asarray(_LF), jnp.asarray(_PIX),
        jnp.asarray(_ZER), jnp.asarray(_DUM))

    band = band.reshape(_NC, _B, _P)
    rec = (dense + band[0] + band[1]).reshape(_B, _H, _W)
    return jnp.clip(rec, 0.0, rec.max())
